# Initial kernel scaffold; baseline (speedup 1.0000x reference)
import jax
import jax.numpy as jnp
from jax.experimental import pallas as pl


def kernel(x, table):
    raise NotImplementedError("write your pallas kernel here")


# speedup vs baseline: 1.3020x; 1.3020x over previous
import functools

import jax
import jax.numpy as jnp
from jax import lax
from jax.experimental import pallas as pl
from jax.experimental.pallas import tpu as pltpu
from jax.experimental.pallas import tpu_sc as plsc

BATCH = 4096
HIST = 50
D_MODEL = 512
TOTAL = BATCH * HIST

NUM_CORES = 2
NUM_SUBCORES = 16
NUM_WORKERS = NUM_CORES * NUM_SUBCORES
ROWS_PER_W = TOTAL // NUM_WORKERS

CHUNK = 80
NBUF = 2
NCHUNK = ROWS_PER_W // CHUNK


def _emb_body(idx_hbm, table_hbm, out_hbm, idx_v, rows0, rows1, sem0, sem1):
    wid = lax.axis_index("s") * NUM_CORES + lax.axis_index("c")
    base = wid * ROWS_PER_W

    pltpu.sync_copy(idx_hbm.at[pl.ds(base, ROWS_PER_W)], idx_v)

    bufs = (rows0, rows1)
    sems = (sem0, sem1)

    def _start(c, b):
        pltpu.async_copy(
            table_hbm.at[idx_v.at[pl.ds(c * CHUNK, CHUNK)]], bufs[b], sems[b]
        )

    def _wait(c, b):
        pltpu.make_async_copy(
            table_hbm.at[idx_v.at[pl.ds(c * CHUNK, CHUNK)]], bufs[b], sems[b]
        ).wait()

    for b in range(NBUF):
        _start(b, b)

    def _group(g, carry):
        c0 = g * NBUF
        for b in range(NBUF):
            c = c0 + b
            _wait(c, b)
            pltpu.sync_copy(bufs[b], out_hbm.at[pl.ds(base + c * CHUNK, CHUNK)])
            _start(c + NBUF, b)
        return carry

    lax.fori_loop(0, (NCHUNK - NBUF) // NBUF, _group, 0)

    for b in range(NBUF):
        c = NCHUNK - NBUF + b
        _wait(c, b)
        pltpu.sync_copy(bufs[b], out_hbm.at[pl.ds(base + c * CHUNK, CHUNK)])


@functools.partial(jax.jit, static_argnames=())
def _embed(idx_flat, table):
    mesh = plsc.VectorSubcoreMesh(core_axis_name="c", subcore_axis_name="s")
    run = pl.kernel(
        _emb_body,
        mesh=mesh,
        out_type=jax.ShapeDtypeStruct((TOTAL, D_MODEL), jnp.float32),
        scratch_types=[
            pltpu.VMEM((ROWS_PER_W,), jnp.int32),
            pltpu.VMEM((CHUNK, D_MODEL), jnp.float32),
            pltpu.VMEM((CHUNK, D_MODEL), jnp.float32),
            pltpu.SemaphoreType.DMA,
            pltpu.SemaphoreType.DMA,
        ],
    )
    return run(idx_flat, table)


def kernel(x, table):
    idx_flat = x.reshape(-1).astype(jnp.int32)
    out = _embed(idx_flat, table)
    return out.reshape(BATCH, HIST, D_MODEL)


# speedup vs baseline: 1.3032x; 1.0009x over previous
import functools

import jax
import jax.numpy as jnp
from jax import lax
from jax.experimental import pallas as pl
from jax.experimental.pallas import tpu as pltpu
from jax.experimental.pallas import tpu_sc as plsc

BATCH = 4096
HIST = 50
D_MODEL = 512
TOTAL = BATCH * HIST

NUM_CORES = 2
NUM_SUBCORES = 16
NUM_WORKERS = NUM_CORES * NUM_SUBCORES
ROWS_PER_W = TOTAL // NUM_WORKERS

CHUNK = 40
NBUF = 4
LOOKAHEAD = 2
NCHUNK = ROWS_PER_W // CHUNK
NGROUP = NCHUNK // NBUF


def _emb_body(idx_hbm, table_hbm, out_hbm,
              idx_v, b0, b1, b2, b3,
              g0, g1, g2, g3, s0, s1, s2, s3):
    wid = lax.axis_index("s") * NUM_CORES + lax.axis_index("c")
    base = wid * ROWS_PER_W

    pltpu.sync_copy(idx_hbm.at[pl.ds(base, ROWS_PER_W)], idx_v)

    bufs = (b0, b1, b2, b3)
    gsems = (g0, g1, g2, g3)
    ssems = (s0, s1, s2, s3)

    def _gather(c, b):
        return pltpu.make_async_copy(
            table_hbm.at[idx_v.at[pl.ds(c * CHUNK, CHUNK)]], bufs[b], gsems[b]
        )

    def _store(c, b):
        return pltpu.make_async_copy(
            bufs[b], out_hbm.at[pl.ds(base + c * CHUNK, CHUNK)], ssems[b]
        )

    for b in range(LOOKAHEAD):
        _gather(b, b).start()

    def _group(g, carry):
        c0 = g * NBUF
        for b in range(NBUF):
            c = c0 + b
            _gather(c, b).wait()
            _store(c, b).start()

            bp = (b - LOOKAHEAD) % NBUF

            @pl.when(c >= LOOKAHEAD)
            def _():
                _store(c - LOOKAHEAD, bp).wait()

            @pl.when(c + LOOKAHEAD < NCHUNK)
            def _():
                _gather(c + LOOKAHEAD, bp).start()

        return carry

    lax.fori_loop(0, NGROUP, _group, 0)

    for k in range(LOOKAHEAD):
        c = NCHUNK - LOOKAHEAD + k
        _store(c, c % NBUF).wait()


@jax.jit
def _embed(idx_flat, table):
    mesh = plsc.VectorSubcoreMesh(core_axis_name="c", subcore_axis_name="s")
    run = pl.kernel(
        _emb_body,
        mesh=mesh,
        out_type=jax.ShapeDtypeStruct((TOTAL, D_MODEL), jnp.float32),
        scratch_types=[
            pltpu.VMEM((ROWS_PER_W,), jnp.int32),
            pltpu.VMEM((CHUNK, D_MODEL), jnp.float32),
            pltpu.VMEM((CHUNK, D_MODEL), jnp.float32),
            pltpu.VMEM((CHUNK, D_MODEL), jnp.float32),
            pltpu.VMEM((CHUNK, D_MODEL), jnp.float32),
            pltpu.SemaphoreType.DMA,
            pltpu.SemaphoreType.DMA,
            pltpu.SemaphoreType.DMA,
            pltpu.SemaphoreType.DMA,
            pltpu.SemaphoreType.DMA,
            pltpu.SemaphoreType.DMA,
            pltpu.SemaphoreType.DMA,
            pltpu.SemaphoreType.DMA,
        ],
    )
    return run(idx_flat, table)


def kernel(x, table):
    idx_flat = x.reshape(-1).astype(jnp.int32)
    out = _embed(idx_flat, table)
    return out.reshape(BATCH, HIST, D_MODEL)


# speedup vs baseline: 1.8516x; 1.4208x over previous
import jax
import jax.numpy as jnp
from jax import lax
from jax.experimental import pallas as pl
from jax.experimental.pallas import tpu as pltpu
from jax.experimental.pallas import tpu_sc as plsc

BATCH = 4096
HIST = 50
HIST_PAD = 56
D_MODEL = 512

NUM_CORES = 2
NUM_SUBCORES = 16
NUM_WORKERS = NUM_CORES * NUM_SUBCORES
B_PER_W = BATCH // NUM_WORKERS

NSLOT = 3
LANES = 16


def _emb_body(idx_hbm, table_hbm, out_hbm,
              idx_v, b0, b1, b2, t0, t1, t2,
              g0, g1, g2, s0, s1, s2):
    wid = lax.axis_index("s") * NUM_CORES + lax.axis_index("c")
    base = wid * B_PER_W

    pltpu.sync_copy(idx_hbm.at[pl.ds(base * HIST_PAD, B_PER_W * HIST_PAD)], idx_v)

    bufs = (b0, b1, b2)
    tails = (t0, t1, t2)
    gsems = (g0, g1, g2)
    ssems = (s0, s1, s2)

    def _gathers(k, b):
        off = k * HIST_PAD
        return (
            pltpu.make_async_copy(
                table_hbm.at[idx_v.at[pl.ds(off, 48)]],
                bufs[b].at[pl.ds(0, 48)], gsems[b]),
            pltpu.make_async_copy(
                table_hbm.at[idx_v.at[pl.ds(off + 48, 8)]],
                tails[b], gsems[b]),
        )

    def _store(k, b):
        return pltpu.make_async_copy(bufs[b], out_hbm.at[base + k], ssems[b])

    for k in range(2):
        for op in _gathers(k, k):
            op.start()

    def _iter(k, carry):
        slot = lax.rem(k, NSLOT)

        def _run(b):
            for op in _gathers(k, b):
                op.wait()
            for r in range(2):
                for c in range(D_MODEL // LANES):
                    bufs[b][48 + r, pl.ds(c * LANES, LANES)] = (
                        tails[b][r, pl.ds(c * LANES, LANES)])
            _store(k, b).start()

            bn = (b + 2) % NSLOT

            @pl.when(k >= 1)
            def _():
                _store(k - 1, bn).wait()

            @pl.when(k + 2 < B_PER_W)
            def _():
                for op in _gathers(k + 2, bn):
                    op.start()

        for b in range(NSLOT):
            @pl.when(slot == b)
            def _(b=b):
                _run(b)

        return carry

    lax.fori_loop(0, B_PER_W, _iter, 0)

    _store(B_PER_W - 1, (B_PER_W - 1) % NSLOT).wait()


@jax.jit
def _embed(idx_pad, table):
    mesh = plsc.VectorSubcoreMesh(core_axis_name="c", subcore_axis_name="s")
    run = pl.kernel(
        _emb_body,
        mesh=mesh,
        out_type=jax.ShapeDtypeStruct((BATCH, HIST, D_MODEL), jnp.float32),
        scratch_types=[
            pltpu.VMEM((B_PER_W * HIST_PAD,), jnp.int32),
            pltpu.VMEM((HIST, D_MODEL), jnp.float32),
            pltpu.VMEM((HIST, D_MODEL), jnp.float32),
            pltpu.VMEM((HIST, D_MODEL), jnp.float32),
            pltpu.VMEM((8, D_MODEL), jnp.float32),
            pltpu.VMEM((8, D_MODEL), jnp.float32),
            pltpu.VMEM((8, D_MODEL), jnp.float32),
            pltpu.SemaphoreType.DMA,
            pltpu.SemaphoreType.DMA,
            pltpu.SemaphoreType.DMA,
            pltpu.SemaphoreType.DMA,
            pltpu.SemaphoreType.DMA,
            pltpu.SemaphoreType.DMA,
        ],
    )
    return run(idx_pad, table)


def kernel(x, table):
    idx_pad = jnp.pad(x.astype(jnp.int32), ((0, 0), (0, HIST_PAD - HIST)),
                      mode="edge")
    return _embed(idx_pad.reshape(-1), table)
